# Initial kernel scaffold; baseline (speedup 1.0000x reference)
import jax
import jax.numpy as jnp
from jax.experimental import pallas as pl


def kernel(node_features, edge_index, edge_distances, triplet_edge_index, triplet_angles, edge_attr, W_lin, b_lin, Wg_main, bg_main, Wg_gate, bg_gate):
    raise NotImplementedError("write your pallas kernel here")


# speedup vs baseline: 32.9263x; 32.9263x over previous
import functools
import math

import jax
import jax.numpy as jnp
from jax import lax
from jax.experimental import pallas as pl
from jax.experimental.pallas import tpu as pltpu
from jax.experimental.pallas import tpu_sc as plsc

CUT = 5.0
EPS = 1e-8
N_NODES = 10000
N_EDGES = 320000
N_TRIPLETS = 2000000
D_NODE = 128
D_EDGE = 16
DEG = 9

_SBZ = (
    (3.141592653589793, 6.283185307179586, 9.42477796076938),
    (4.4934094579080615, 7.725251836938652, 10.904121659429897),
    (5.763459196895549, 9.09501133047736, 12.322940970567323),
)
_C0 = math.sqrt(1.0 / math.pi)
_C1 = math.sqrt(3.0 / math.pi)
_C2 = math.sqrt(5.0 / math.pi)

NCORES = 2
NSUB = 16
NW = NCORES * NSUB
E_PAD = 327680
E_HALF = E_PAD // NCORES
TRASH = 128
ACC_ROWS = E_HALF + TRASH

T_PAD = 2031616
TPW = 512
NCHUNK = 4
WINDOWS = 248
TROWS_PER_TILE = WINDOWS * NCHUNK

E_PER_TILE = E_PAD // NW
ECHUNK = 2048
ECHUNKS = E_PER_TILE // ECHUNK
DCHUNK = 256
DCHUNKS = E_PER_TILE // DCHUNK
INIT_ROWS_PER_TILE = ACC_ROWS // NSUB


def _midnode_body(x_ref, w_ref, b_ref, o_ref):
    acc = jnp.dot(x_ref[...], w_ref[...], preferred_element_type=jnp.float32,
                  precision=lax.Precision.HIGHEST)
    o_ref[...] = jax.nn.sigmoid(acc + b_ref[...])


def _midnode(node_features, w16, b16):
    blk = 2000
    return pl.pallas_call(
        _midnode_body,
        grid=(N_NODES // blk,),
        in_specs=[
            pl.BlockSpec((blk, D_NODE), lambda i: (i, 0)),
            pl.BlockSpec((D_NODE, 16), lambda i: (0, 0)),
            pl.BlockSpec((1, 16), lambda i: (0, 0)),
        ],
        out_specs=pl.BlockSpec((blk, 16), lambda i: (i, 0)),
        out_shape=jax.ShapeDtypeStruct((N_NODES, 16), jnp.float32),
    )(node_features, w16, b16)


def _basis_body(r_ref, bfc_ref, fc_ref):
    r = r_ref[...]
    ratio = r / CUT
    fc = 1 - 6 * ratio ** 5 + 15 * ratio ** 4 - 10 * ratio ** 3
    fc_ref[...] = fc
    for l in range(3):
        for n in range(3):
            x = _SBZ[l][n] * r / CUT
            safe = x > EPS
            xs = jnp.where(safe, x, jnp.ones_like(x))
            j0 = jnp.where(safe, jnp.sin(xs) / xs, jnp.ones_like(x))
            if l == 0:
                j = j0
            else:
                j1 = jnp.where(safe, (jnp.sin(xs) / xs - jnp.cos(xs)) / xs,
                               x / 3.0)
                if l == 1:
                    j = j1
                else:
                    j = jnp.where(safe, 3.0 / xs * j1 - j0, x / 15.0)
            bfc_ref[3 * l + n] = j * fc
    for d in range(DEG, 16):
        bfc_ref[d] = jnp.zeros_like(r)


def _basis(r2d):
    rows, cols = r2d.shape
    blk = 256
    return pl.pallas_call(
        _basis_body,
        grid=(rows // blk,),
        in_specs=[pl.BlockSpec((blk, cols), lambda i: (i, 0))],
        out_specs=[
            pl.BlockSpec((16, blk, cols), lambda i: (0, i, 0)),
            pl.BlockSpec((blk, cols), lambda i: (i, 0)),
        ],
        out_shape=[
            jax.ShapeDtypeStruct((16, rows, cols), jnp.float32),
            jax.ShapeDtypeStruct((rows, cols), jnp.float32),
        ],
    )(r2d)


def _sc_build_c(mid_node_ref, bfct_ref, k2d_ref, c_ref,
                kchunk, nk, bfcc, cout, sem):
    cid = lax.axis_index("c")
    sid = lax.axis_index("s")
    wid = sid * NCORES + cid
    lanes = lax.iota(jnp.int32, 16)
    nrow = ECHUNK // 128

    def zinit(i, _):
        cout[i] = jnp.zeros((16,), jnp.float32)
        return _
    lax.fori_loop(0, ECHUNK, zinit, None)

    for c in range(ECHUNKS):
        ebase = pl.multiple_of(wid * E_PER_TILE + c * ECHUNK, ECHUNK)
        krow = pl.multiple_of(ebase // 128, nrow)
        pltpu.sync_copy(k2d_ref.at[pl.ds(krow, nrow)], kchunk)
        copies = []
        for j in range(nrow):
            cp = pltpu.async_copy(
                mid_node_ref.at[kchunk.at[j]],
                nk.at[pl.ds(j * 128, 128)], sem)
            copies.append(cp)
        pltpu.sync_copy(bfct_ref.at[:, pl.ds(krow, nrow), :], bfcc)
        for cp in copies:
            cp.wait()

        def egroup(eg, _):
            j = eg >> 3
            col = pl.multiple_of((eg & 7) * 16, 16)
            e_idx = jnp.full((16,), eg * 16, jnp.int32) + lanes
            for d in range(DEG):
                dv = jnp.full((16,), d, jnp.int32)
                b = bfcc[d, j, pl.ds(col, 16)]
                nkv = plsc.load_gather(nk, [e_idx, dv])
                plsc.store_scatter(cout, [e_idx, dv], b * nkv)
            return _
        lax.fori_loop(0, ECHUNK // 16, egroup, None)
        pltpu.sync_copy(cout, c_ref.at[pl.ds(ebase, ECHUNK)])


def _build_c(mid_node, bfct, k2d):
    mesh = plsc.VectorSubcoreMesh(core_axis_name="c", subcore_axis_name="s")
    f = functools.partial(
        pl.kernel, mesh=mesh,
        out_type=jax.ShapeDtypeStruct((E_PAD, 16), jnp.float32),
        compiler_params=pltpu.CompilerParams(
            use_tc_tiling_on_sc=False, needs_layout_passes=False),
        scratch_types=[
            pltpu.VMEM((ECHUNK // 128, 128), jnp.int32),
            pltpu.VMEM((ECHUNK, 16), jnp.float32),
            pltpu.VMEM((16, ECHUNK // 128, 128), jnp.float32),
            pltpu.VMEM((ECHUNK, 16), jnp.float32),
            pltpu.SemaphoreType.DMA,
        ],
    )(_sc_build_c)
    return f(mid_node, bfct, k2d)


def _sc_triplet(c_ref, t0_ref, t1_ref, ang_ref, fc_ref, me_ref,
                t0v, t1v, angv, dstv, rows, upd8, upd1,
                acc8chunk, acc1chunk, outchunk, fcv,
                gsem, ssem, acc8, acc1):
    cid = lax.axis_index("c")
    sid = lax.axis_index("s")
    lanes = lax.iota(jnp.int32, 16)
    core_base = cid * E_HALF

    zero16 = jnp.zeros((16,), jnp.float32)

    def zinit(i, _):
        w = jnp.full((16,), i * 16, jnp.int32) + lanes
        rv = w >> 3
        cv = w & 7
        plsc.store_scatter(acc8chunk, [rv, cv], zero16)
        acc1chunk[pl.ds(pl.multiple_of(i * 16 % DCHUNK, 16), 16)] = zero16
        return _
    lax.fori_loop(0, DCHUNK * 8 // 16, zinit, None)
    irow = sid * INIT_ROWS_PER_TILE
    for k in range(INIT_ROWS_PER_TILE // DCHUNK):
        pltpu.sync_copy(acc8chunk, acc8.at[pl.ds(irow + k * DCHUNK, DCHUNK)])
        pltpu.sync_copy(acc1chunk, acc1.at[pl.ds(irow + k * DCHUNK, DCHUNK)])
    rem = INIT_ROWS_PER_TILE % DCHUNK
    if rem:
        pltpu.sync_copy(acc8chunk.at[pl.ds(0, rem)],
                        acc8.at[pl.ds(irow + INIT_ROWS_PER_TILE - rem, rem)])
        pltpu.sync_copy(acc1chunk.at[pl.ds(0, rem)],
                        acc1.at[pl.ds(irow + INIT_ROWS_PER_TILE - rem, rem)])

    def zinit2(i, _):
        outchunk[i] = jnp.zeros((16,), jnp.float32)
        return _
    lax.fori_loop(0, DCHUNK, zinit2, None)

    plsc.subcore_barrier()

    def window(w, _):
        rbase = pl.multiple_of(sid * TROWS_PER_TILE + w * NCHUNK, NCHUNK)
        pltpu.sync_copy(t0_ref.at[pl.ds(rbase, NCHUNK)], t0v)
        pltpu.sync_copy(t1_ref.at[pl.ds(rbase, NCHUNK)], t1v)
        pltpu.sync_copy(ang_ref.at[pl.ds(rbase, NCHUNK)], angv)
        gcopies = []
        for j in range(NCHUNK):
            cp = pltpu.async_copy(
                c_ref.at[t1v.at[j]], rows.at[pl.ds(j * 128, 128)], gsem)
            gcopies.append(cp)
        for cp in gcopies:
            cp.wait()

        def group(g, _):
            j = g >> 3
            col = pl.multiple_of((g & 7) * 16, 16)
            a = angv[j, pl.ds(col, 16)]
            e0 = t0v[j, pl.ds(col, 16)]
            local = e0 - core_base
            inr = (local >= 0) & (local < E_HALF)
            dst = jnp.where(inr, local,
                            E_HALF + ((jnp.full((16,), g, jnp.int32) + lanes)
                                      & (TRASH - 1)))
            dstv[j, pl.ds(col, 16)] = dst
            mulB = _C1 * a
            mulC = _C2 * (1.5 * a * a - 0.5)
            mulA = jnp.full((16,), _C0, jnp.float32)
            t_idx = jnp.full((16,), g * 16, jnp.int32) + lanes
            jv = jnp.full((16,), 0, jnp.int32) + j
            rv = jnp.full((16,), col, jnp.int32) + lanes
            for d in range(8):
                dv = jnp.full((16,), d, jnp.int32)
                cval = plsc.load_gather(rows, [t_idx, dv])
                mul = (mulA, mulB, mulC)[d // 3]
                plsc.store_scatter(upd8, [jv, rv, dv], cval * mul)
            c8 = plsc.load_gather(rows, [t_idx, jnp.full((16,), 8, jnp.int32)])
            plsc.store_scatter(upd1, [jv, rv], c8 * mulC)
            return _
        lax.fori_loop(0, TPW // 16, group, None)

        scopies = []
        for j in range(NCHUNK):
            scopies.append(pltpu.async_copy(
                upd8.at[j], acc8.at[dstv.at[j]], ssem, add=True))
            scopies.append(pltpu.async_copy(
                upd1.at[j], acc1.at[dstv.at[j]], ssem, add=True))
        for cp in scopies:
            cp.wait()
        return _
    lax.fori_loop(0, WINDOWS, window, None)

    plsc.subcore_barrier()

    for c in range(DCHUNKS):
        lbase = pl.multiple_of(sid * E_PER_TILE + c * DCHUNK, DCHUNK)
        gbase = pl.multiple_of(core_base + lbase, DCHUNK)
        pltpu.sync_copy(acc8.at[pl.ds(lbase, DCHUNK)], acc8chunk)
        pltpu.sync_copy(acc1.at[pl.ds(lbase, DCHUNK)], acc1chunk)
        pltpu.sync_copy(fc_ref.at[pl.ds(gbase, DCHUNK)], fcv)

        def dgroup(eg, _):
            e_idx = jnp.full((16,), eg * 16, jnp.int32) + lanes
            off = pl.multiple_of(eg * 16, 16)
            f = fcv[pl.ds(off, 16)]
            for d in range(8):
                dv = jnp.full((16,), d, jnp.int32)
                v = plsc.load_gather(acc8chunk, [e_idx, dv])
                plsc.store_scatter(outchunk, [e_idx, dv], v * f)
            v8 = acc1chunk[pl.ds(off, 16)]
            plsc.store_scatter(outchunk, [e_idx, jnp.full((16,), 8, jnp.int32)],
                               v8 * f)
            return _
        lax.fori_loop(0, DCHUNK // 16, dgroup, None)
        pltpu.sync_copy(outchunk, me_ref.at[pl.ds(gbase, DCHUNK)])


def _triplet_stage(c_tab, t02d, t12d, ang2d, fc_flat):
    mesh = plsc.VectorSubcoreMesh(core_axis_name="c", subcore_axis_name="s")
    f = functools.partial(
        pl.kernel, mesh=mesh,
        out_type=jax.ShapeDtypeStruct((E_PAD, 16), jnp.float32),
        compiler_params=pltpu.CompilerParams(
            use_tc_tiling_on_sc=False, needs_layout_passes=False),
        scratch_types=[
            pltpu.VMEM((NCHUNK, 128), jnp.int32),
            pltpu.VMEM((NCHUNK, 128), jnp.int32),
            pltpu.VMEM((NCHUNK, 128), jnp.float32),
            pltpu.VMEM((NCHUNK, 128), jnp.int32),
            pltpu.VMEM((TPW, 16), jnp.float32),
            pltpu.VMEM((NCHUNK, 128, 8), jnp.float32),
            pltpu.VMEM((NCHUNK, 128), jnp.float32),
            pltpu.VMEM((DCHUNK, 8), jnp.float32),
            pltpu.VMEM((DCHUNK,), jnp.float32),
            pltpu.VMEM((DCHUNK, 16), jnp.float32),
            pltpu.VMEM((DCHUNK,), jnp.float32),
            pltpu.SemaphoreType.DMA,
            pltpu.SemaphoreType.DMA,
            pltpu.VMEM_SHARED((ACC_ROWS, 8), jnp.float32),
            pltpu.VMEM_SHARED((ACC_ROWS,), jnp.float32),
        ],
    )(_sc_triplet)
    return f(c_tab, t02d, t12d, ang2d, fc_flat)


def _mlp_body(me_ref, ea_ref, wm_ref, bm_ref, wg_ref, bg_ref, o_ref):
    me = me_ref[...]
    h = jnp.dot(me, wm_ref[...], preferred_element_type=jnp.float32,
                precision=lax.Precision.HIGHEST) + bm_ref[...]
    g = jnp.dot(me, wg_ref[...], preferred_element_type=jnp.float32,
                precision=lax.Precision.HIGHEST) + bg_ref[...]
    o_ref[...] = ea_ref[...] + (h * jax.nn.sigmoid(h)) * jax.nn.sigmoid(g)


def _mlp(me, edge_attr, wm16, bm16, wg16, bg16):
    blk = 8000
    return pl.pallas_call(
        _mlp_body,
        grid=(N_EDGES // blk,),
        in_specs=[
            pl.BlockSpec((blk, 16), lambda i: (i, 0)),
            pl.BlockSpec((blk, D_EDGE), lambda i: (i, 0)),
            pl.BlockSpec((16, D_EDGE), lambda i: (0, 0)),
            pl.BlockSpec((1, D_EDGE), lambda i: (0, 0)),
            pl.BlockSpec((16, D_EDGE), lambda i: (0, 0)),
            pl.BlockSpec((1, D_EDGE), lambda i: (0, 0)),
        ],
        out_specs=pl.BlockSpec((blk, D_EDGE), lambda i: (i, 0)),
        out_shape=jax.ShapeDtypeStruct((N_EDGES, D_EDGE), jnp.float32),
    )(me, edge_attr, wm16, bm16, wg16, bg16)


def kernel(node_features, edge_index, edge_distances, triplet_edge_index,
           triplet_angles, edge_attr, W_lin, b_lin, Wg_main, bg_main,
           Wg_gate, bg_gate):
    f32 = jnp.float32
    w16 = jnp.pad(W_lin.astype(f32), ((0, 0), (0, 16 - DEG)))
    b16 = jnp.pad(b_lin.astype(f32), (0, 16 - DEG)).reshape(1, 16)
    mid_node = _midnode(node_features.astype(f32), w16, b16)

    epad = E_PAD - N_EDGES
    r2d = jnp.pad(edge_distances.astype(f32), (0, epad)).reshape(E_PAD // 128, 128)
    bfct, fc2d = _basis(r2d)
    fc_flat = fc2d.reshape(E_PAD)

    k2d = jnp.pad(edge_index[1].astype(jnp.int32), (0, epad)).reshape(
        E_PAD // 128, 128)
    c_tab = _build_c(mid_node, bfct, k2d)

    npad = T_PAD - N_TRIPLETS
    t0 = jnp.concatenate([triplet_edge_index[0].astype(jnp.int32),
                          jnp.full((npad,), N_EDGES, jnp.int32)])
    t1 = jnp.concatenate([triplet_edge_index[1].astype(jnp.int32),
                          jnp.zeros((npad,), jnp.int32)])
    ang = jnp.concatenate([triplet_angles.astype(f32), jnp.zeros((npad,), f32)])
    t02d = t0.reshape(T_PAD // 128, 128)
    t12d = t1.reshape(T_PAD // 128, 128)
    ang2d = ang.reshape(T_PAD // 128, 128)

    me = _triplet_stage(c_tab, t02d, t12d, ang2d, fc_flat)[:N_EDGES]

    wm16 = jnp.pad(Wg_main.astype(f32), ((0, 16 - DEG), (0, 0)))
    wg16 = jnp.pad(Wg_gate.astype(f32), ((0, 16 - DEG), (0, 0)))
    bm16 = bg_main.astype(f32).reshape(1, D_EDGE)
    bg16 = bg_gate.astype(f32).reshape(1, D_EDGE)
    return _mlp(me, edge_attr.astype(f32), wm16, bm16, wg16, bg16)
